# baseline (speedup 1.0000x reference)
import functools

import jax
import jax.numpy as jnp
from jax import lax
from jax.experimental import pallas as pl
from jax.experimental.pallas import tpu as pltpu
from jax.experimental.pallas import tpu_sc as plsc

B, S = 64, 512
N = B * S
D = 128
EPS = 1e-12
NC, NS = 2, 16
NW = NC * NS
TOK_PER_W = N // NW
W = 128
NWIN = TOK_PER_W // W
LANES = 16


def _sc_gather_sum(word, pos_t, tok_t, wids, pids, tids):
    mesh = plsc.VectorSubcoreMesh(core_axis_name="c", subcore_axis_name="s")

    @functools.partial(
        pl.kernel,
        mesh=mesh,
        out_type=jax.ShapeDtypeStruct((N, D), jnp.float32),
        scratch_types=[
            pltpu.VMEM((W,), jnp.int32),
            pltpu.VMEM((W,), jnp.int32),
            pltpu.VMEM((W,), jnp.int32),
            pltpu.VMEM((W, D), jnp.float32),
            pltpu.VMEM((W, D), jnp.float32),
            pltpu.VMEM((W, D), jnp.float32),
            pltpu.SemaphoreType.DMA,
            pltpu.SemaphoreType.DMA,
            pltpu.SemaphoreType.DMA,
        ],
    )
    def k(word_hbm, pos_hbm, tok_hbm, wid_hbm, pid_hbm, tid_hbm, out_hbm,
          widx_v, pidx_v, tidx_v, wrow_v, prow_v, trow_v, sem0, sem1, sem2):
        w_id = lax.axis_index("s") * NC + lax.axis_index("c")
        base = w_id * TOK_PER_W

        @pl.loop(0, NWIN)
        def _win_loop(win):
            tok_base = base + win * W
            pltpu.sync_copy(wid_hbm.at[pl.ds(tok_base, W)], widx_v)
            pltpu.sync_copy(pid_hbm.at[pl.ds(tok_base, W)], pidx_v)
            pltpu.sync_copy(tid_hbm.at[pl.ds(tok_base, W)], tidx_v)
            cw = pltpu.async_copy(word_hbm.at[widx_v], wrow_v, sem0)
            cp = pltpu.async_copy(pos_hbm.at[pidx_v], prow_v, sem1)
            ct = pltpu.async_copy(tok_hbm.at[tidx_v], trow_v, sem2)
            cw.wait()
            cp.wait()
            ct.wait()

            @pl.loop(0, W)
            def _row_loop(r):
                for j in range(D // LANES):
                    slc = (pl.ds(r, 1), pl.ds(j * LANES, LANES))
                    wrow_v.at[slc[0], slc[1]][...] = (
                        wrow_v.at[slc[0], slc[1]][...]
                        + prow_v.at[slc[0], slc[1]][...]
                        + trow_v.at[slc[0], slc[1]][...]
                    )

            pltpu.sync_copy(wrow_v, out_hbm.at[pl.ds(tok_base, W)])

    return k(word, pos_t, tok_t, wids, pids, tids)


def _tc_layernorm(summed, gamma, beta):
    rows = 2048

    def body(x_ref, g_ref, b_ref, o_ref):
        x = x_ref[...]
        mean = jnp.mean(x, axis=-1, keepdims=True)
        xc = x - mean
        var = jnp.mean(xc * xc, axis=-1, keepdims=True)
        o_ref[...] = xc * lax.rsqrt(var + EPS) * g_ref[...] + b_ref[...]

    return pl.pallas_call(
        body,
        grid=(N // rows,),
        in_specs=[
            pl.BlockSpec((rows, D), lambda i: (i, 0)),
            pl.BlockSpec((1, D), lambda i: (0, 0)),
            pl.BlockSpec((1, D), lambda i: (0, 0)),
        ],
        out_specs=pl.BlockSpec((rows, D), lambda i: (i, 0)),
        out_shape=jax.ShapeDtypeStruct((N, D), jnp.float32),
    )(summed, gamma.reshape(1, D), beta.reshape(1, D))


def kernel(input_ids, position_ids, token_type_ids, word_embeddings,
           position_embeddings, token_type_embeddings, gamma, beta):
    wids = input_ids.reshape(-1).astype(jnp.int32)
    pids = position_ids.reshape(-1).astype(jnp.int32)
    tids = token_type_ids.reshape(-1).astype(jnp.int32)
    summed = _sc_gather_sum(word_embeddings, position_embeddings,
                            token_type_embeddings, wids, pids, tids)
    out = _tc_layernorm(summed, gamma, beta)
    return out.reshape(B, S, D)


# speedup vs baseline: 9.4205x; 9.4205x over previous
import dataclasses
import functools

import jax
import jax.numpy as jnp
from jax import lax
from jax.experimental import pallas as pl
from jax.experimental.pallas import tpu as pltpu
from jax.experimental.pallas import tpu_sc as plsc

B, S = 64, 512
N = B * S
D = 128
EPS = 1e-12
NC, NS = 2, 16
NW = NC * NS
TOK_PER_W = N // NW
W = 128
NWIN = TOK_PER_W // W
NBUF = 2
LANES = 16
NCH = D // LANES


def _sc_gather_sum(word, pos_t, tok_t, wids, pids, tids):
    mesh = plsc.VectorSubcoreMesh(core_axis_name="c", subcore_axis_name="s")
    cp = pltpu.CompilerParams()
    if "needs_layout_passes" in pltpu.CompilerParams.__dataclass_fields__:
        cp = dataclasses.replace(cp, needs_layout_passes=False)

    @functools.partial(
        pl.kernel,
        mesh=mesh,
        compiler_params=cp,
        out_type=jax.ShapeDtypeStruct((N, D), jnp.float32),
        scratch_types=[
            pltpu.VMEM((TOK_PER_W,), jnp.int32),
            pltpu.VMEM((TOK_PER_W,), jnp.int32),
            pltpu.VMEM((TOK_PER_W,), jnp.int32),
            pltpu.VMEM((2, D), jnp.float32),
            pltpu.VMEM((NBUF, W, D), jnp.float32),
            pltpu.VMEM((NBUF, W, D), jnp.float32),
            pltpu.VMEM((NBUF, W, D), jnp.float32),
            pltpu.SemaphoreType.DMA,
            pltpu.SemaphoreType.DMA,
            pltpu.SemaphoreType.DMA,
            pltpu.SemaphoreType.DMA,
            pltpu.SemaphoreType.DMA,
            pltpu.SemaphoreType.DMA,
        ],
    )
    def k(word_hbm, pos_hbm, tok_hbm, wid_hbm, pid_hbm, tid_hbm, out_hbm,
          widx_v, pidx_v, tidx_v, tok_v, wrow_v, prow_v, srow_v,
          gw0, gw1, gp0, gp1, os0, os1):
        gsemw = (gw0, gw1)
        gsemp = (gp0, gp1)
        osem = (os0, os1)
        w_id = lax.axis_index("s") * NC + lax.axis_index("c")
        base = w_id * TOK_PER_W

        pltpu.sync_copy(wid_hbm.at[pl.ds(base, TOK_PER_W)], widx_v)
        pltpu.sync_copy(pid_hbm.at[pl.ds(base, TOK_PER_W)], pidx_v)
        pltpu.sync_copy(tid_hbm.at[pl.ds(base, TOK_PER_W)], tidx_v)
        pltpu.sync_copy(tok_hbm, tok_v)

        t0c = []
        dtc = []
        for j in range(NCH):
            cs = pl.ds(j * LANES, LANES)
            t0 = tok_v.at[0, cs][...]
            t1 = tok_v.at[1, cs][...]
            t0c.append(t0)
            dtc.append(t1 - t0)

        def issue_gathers(g, b):
            isl = pl.ds(g * W, W)
            pltpu.async_copy(word_hbm.at[widx_v.at[isl]], wrow_v.at[b],
                             gsemw[b])
            pltpu.async_copy(pos_hbm.at[pidx_v.at[isl]], prow_v.at[b],
                             gsemp[b])

        def wait_gathers(b):
            pltpu.make_async_copy(word_hbm.at[pl.ds(0, W)], wrow_v.at[b],
                                  gsemw[b]).wait()
            pltpu.make_async_copy(pos_hbm.at[pl.ds(0, W)], prow_v.at[b],
                                  gsemp[b]).wait()

        def wait_out(b):
            pltpu.make_async_copy(srow_v.at[b], out_hbm.at[pl.ds(base, W)],
                                  osem[b]).wait()

        for b in range(NBUF):
            issue_gathers(b, b)

        @pl.loop(0, NWIN, step=NBUF)
        def _ring(g0):
            for b in range(NBUF):
                g = g0 + b
                @pl.when(g0 > 0)
                def _():
                    wait_out(b)

                wait_gathers(b)
                wb = wrow_v.at[b]
                pb = prow_v.at[b]
                sb = srow_v.at[b]

                @pl.loop(0, W)
                def _row(r):
                    tid = plsc.load_gather(
                        tidx_v, [jnp.full((LANES,), g * W + r, jnp.int32)])
                    tid_f = tid.astype(jnp.float32)
                    for j in range(NCH):
                        cs = pl.ds(j * LANES, LANES)
                        sb.at[r, cs][...] = (
                            wb.at[r, cs][...] + pb.at[r, cs][...]
                            + (t0c[j] + tid_f * dtc[j]))

                pltpu.async_copy(sb, out_hbm.at[pl.ds(base + g * W, W)],
                                 osem[b])

                @pl.when(g + NBUF < NWIN)
                def _():
                    issue_gathers(g + NBUF, b)

        for b in range(NBUF):
            wait_out(b)

    return k(word, pos_t, tok_t, wids, pids, tids)


def _tc_layernorm(summed, gamma, beta):
    rows = 2048

    def body(x_ref, g_ref, b_ref, o_ref):
        x = x_ref[...]
        mean = jnp.mean(x, axis=-1, keepdims=True)
        xc = x - mean
        var = jnp.mean(xc * xc, axis=-1, keepdims=True)
        o_ref[...] = xc * lax.rsqrt(var + EPS) * g_ref[...] + b_ref[...]

    return pl.pallas_call(
        body,
        grid=(N // rows,),
        in_specs=[
            pl.BlockSpec((rows, D), lambda i: (i, 0)),
            pl.BlockSpec((1, D), lambda i: (0, 0)),
            pl.BlockSpec((1, D), lambda i: (0, 0)),
        ],
        out_specs=pl.BlockSpec((rows, D), lambda i: (i, 0)),
        out_shape=jax.ShapeDtypeStruct((N, D), jnp.float32),
    )(summed, gamma.reshape(1, D), beta.reshape(1, D))


def kernel(input_ids, position_ids, token_type_ids, word_embeddings,
           position_embeddings, token_type_embeddings, gamma, beta):
    wids = input_ids.reshape(-1).astype(jnp.int32)
    pids = position_ids.reshape(-1).astype(jnp.int32)
    tids = token_type_ids.reshape(-1).astype(jnp.int32)
    summed = _sc_gather_sum(word_embeddings, position_embeddings,
                            token_type_embeddings, wids, pids, tids)
    out = _tc_layernorm(summed, gamma, beta)
    return out.reshape(B, S, D)
